# baseline (device time: 41122 ns/iter reference)
import jax
import jax.numpy as jnp
from jax import lax
from jax.experimental import pallas as pl
from jax.experimental.pallas import tpu as pltpu

T = 2048
D = 1024
VSH = 16384
Q = 512
C = 4
CH = Q // C


def kernel(ids, E):
    my_x = lax.axis_index("x")
    rpre = ids - my_x * VSH

    def body(r_smem, e_ref, out_ref,
             gbuf, pbuf, xbuf, cbuf, ybuf, zbuf, dbuf, obuf,
             gsem, sx, rx, sy, ry, sz, rz, sfy, rfy, sfz, rfz, osem):
        x = lax.axis_index("x")
        y = lax.axis_index("y")
        z = lax.axis_index("z")
        px = (1 - x, y, z)
        py = (x, 1 - y, z)
        pz = (x, y, 1 - z)
        myq = 2 * y + z
        yq = 2 * (1 - y) + z
        zq = 2 * y + (1 - z)
        dq = 2 * (1 - y) + (1 - z)
        q0 = myq * Q

        barrier = pltpu.get_barrier_semaphore()
        for nbr in (px, py, pz):
            pl.semaphore_signal(
                barrier, inc=1, device_id=nbr,
                device_id_type=pl.DeviceIdType.MESH,
            )

        gbuf[...] = jnp.zeros((Q, D), jnp.float32)

        odmas = []

        def store_out(qidx, c, expr):
            slot = len(odmas)
            obuf[slot, :, :] = expr
            d = pltpu.make_async_copy(
                obuf.at[slot],
                out_ref.at[pl.ds(qidx * Q + c * CH, CH), :],
                osem.at[slot],
            )
            d.start()
            odmas.append(d)

        def gather_chunk(c):
            def one(i, cnt):
                r = r_smem[q0 + c * CH + i]
                in_half = (r >= 0) & (r < VSH)
                rc = jnp.clip(r, 0, VSH - 1)

                @pl.when(in_half)
                def _():
                    pltpu.make_async_copy(
                        e_ref.at[pl.ds(rc, 1), :],
                        gbuf.at[pl.ds(c * CH + i, 1), :],
                        gsem.at[c],
                    ).start()

                return cnt + in_half.astype(jnp.int32)

            def two(k, cnt):
                cnt = one(2 * k, cnt)
                return one(2 * k + 1, cnt)

            return lax.fori_loop(0, CH // 2, two, 0)

        def send_x(c, nc):
            def drain(i, _):
                pltpu.make_async_copy(
                    e_ref.at[pl.ds(0, 1), :], gbuf.at[pl.ds(0, 1), :],
                    gsem.at[c],
                ).wait()
                return 0

            lax.fori_loop(0, nc, drain, 0)
            sl = pl.ds(c * CH, CH)
            pbuf[sl, :] = gbuf[sl, :].astype(jnp.bfloat16)
            r = pltpu.make_async_remote_copy(
                src_ref=pbuf.at[sl, :], dst_ref=xbuf.at[sl, :],
                send_sem=sx.at[c], recv_sem=rx.at[c],
                device_id=px, device_id_type=pl.DeviceIdType.MESH,
            )
            r.start()
            return r

        def complete(c):
            sl = pl.ds(c * CH, CH)
            rdx[c].wait_recv()
            cbuf[sl, :] = pbuf[sl, :] + xbuf[sl, :]
            ry_ = pltpu.make_async_remote_copy(
                src_ref=cbuf.at[sl, :], dst_ref=ybuf.at[sl, :],
                send_sem=sy.at[c], recv_sem=ry.at[c],
                device_id=py, device_id_type=pl.DeviceIdType.MESH,
            )
            ry_.start()
            rz_ = pltpu.make_async_remote_copy(
                src_ref=cbuf.at[sl, :], dst_ref=zbuf.at[sl, :],
                send_sem=sz.at[c], recv_sem=rz.at[c],
                device_id=pz, device_id_type=pl.DeviceIdType.MESH,
            )
            rz_.start()
            rdy.append(ry_)
            rdz.append(rz_)
            store_out(myq, c, cbuf[sl, :].astype(jnp.float32))

        def recv_z(c):
            sl = pl.ds(c * CH, CH)
            rdz[c].wait_recv()
            if c < 2:
                r = pltpu.make_async_remote_copy(
                    src_ref=zbuf.at[sl, :], dst_ref=dbuf.at[sl, :],
                    send_sem=sfy.at[c], recv_sem=rfy.at[c],
                    device_id=py, device_id_type=pl.DeviceIdType.MESH,
                )
                r.start()
                fw.append(r)
            store_out(zq, c, zbuf[sl, :].astype(jnp.float32))

        def recv_y(c):
            sl = pl.ds(c * CH, CH)
            rdy[c].wait_recv()
            if c >= 2:
                r = pltpu.make_async_remote_copy(
                    src_ref=ybuf.at[sl, :], dst_ref=dbuf.at[sl, :],
                    send_sem=sfz.at[c - 2], recv_sem=rfz.at[c - 2],
                    device_id=pz, device_id_type=pl.DeviceIdType.MESH,
                )
                r.start()
                fw.append(r)
            store_out(yq, c, ybuf[sl, :].astype(jnp.float32))

        def recv_d(k):
            sl = pl.ds(k * CH, CH)
            fw[k].wait_recv()
            store_out(dq, k, dbuf[sl, :].astype(jnp.float32))

        rdx, rdy, rdz, fw = [], [], [], []

        n0 = gather_chunk(0)
        pl.semaphore_wait(barrier, 3)
        rdx.append(send_x(0, n0))
        n1 = gather_chunk(1)
        rdx.append(send_x(1, n1))
        n2 = gather_chunk(2)
        complete(0)
        rdx.append(send_x(2, n2))
        n3 = gather_chunk(3)
        complete(1)
        rdx.append(send_x(3, n3))
        recv_z(0)
        complete(2)
        recv_y(0)
        recv_z(1)
        complete(3)
        recv_y(1)
        recv_z(2)
        recv_y(2)
        recv_z(3)
        recv_y(3)
        recv_d(0)
        recv_d(1)
        recv_d(2)
        recv_d(3)

        for r in rdx + rdy + rdz + fw:
            r.wait_send()
        for d in odmas:
            d.wait()

    return pl.pallas_call(
        body,
        out_shape=jax.ShapeDtypeStruct((T, D), jnp.float32),
        in_specs=[
            pl.BlockSpec(memory_space=pltpu.SMEM),
            pl.BlockSpec(memory_space=pl.ANY),
        ],
        out_specs=pl.BlockSpec(memory_space=pl.ANY),
        scratch_shapes=[
            pltpu.VMEM((Q, D), jnp.float32),
            pltpu.VMEM((Q, D), jnp.bfloat16),
            pltpu.VMEM((Q, D), jnp.bfloat16),
            pltpu.VMEM((Q, D), jnp.bfloat16),
            pltpu.VMEM((Q, D), jnp.bfloat16),
            pltpu.VMEM((Q, D), jnp.bfloat16),
            pltpu.VMEM((Q, D), jnp.bfloat16),
            pltpu.VMEM((4 * C, CH, D), jnp.float32),
            pltpu.SemaphoreType.DMA((C,)),
            pltpu.SemaphoreType.DMA((C,)),
            pltpu.SemaphoreType.DMA((C,)),
            pltpu.SemaphoreType.DMA((C,)),
            pltpu.SemaphoreType.DMA((C,)),
            pltpu.SemaphoreType.DMA((C,)),
            pltpu.SemaphoreType.DMA((C,)),
            pltpu.SemaphoreType.DMA((2,)),
            pltpu.SemaphoreType.DMA((2,)),
            pltpu.SemaphoreType.DMA((2,)),
            pltpu.SemaphoreType.DMA((2,)),
            pltpu.SemaphoreType.DMA((4 * C,)),
        ],
        compiler_params=pltpu.CompilerParams(collective_id=0),
    )(rpre, E)


# device time: 39776 ns/iter; 1.0338x vs baseline; 1.0338x over previous
import jax
import jax.numpy as jnp
from jax import lax
from jax.experimental import pallas as pl
from jax.experimental.pallas import tpu as pltpu

T = 2048
D = 1024
VSH = 16384
Q = 512
C = 4
CH = Q // C


def kernel(ids, E):
    my_x = lax.axis_index("x")
    rpre = ids - my_x * VSH

    def body(r_smem, e_ref, out_ref,
             gbuf, pbuf, xbuf, cbuf, ybuf, zbuf, dbuf,
             gsem, sx, rx, sy, ry, sz, rz, sfy, rfy, sfz, rfz, osem):
        x = lax.axis_index("x")
        y = lax.axis_index("y")
        z = lax.axis_index("z")
        px = (1 - x, y, z)
        py = (x, 1 - y, z)
        pz = (x, y, 1 - z)
        myq = 2 * y + z
        yq = 2 * (1 - y) + z
        zq = 2 * y + (1 - z)
        dq = 2 * (1 - y) + (1 - z)
        q0 = myq * Q

        barrier = pltpu.get_barrier_semaphore()
        for nbr in (px, py, pz):
            pl.semaphore_signal(
                barrier, inc=1, device_id=nbr,
                device_id_type=pl.DeviceIdType.MESH,
            )

        gbuf[...] = jnp.zeros((Q, D), jnp.float32)

        odmas = []

        def store_out(qidx, c, srcbuf):
            slot = len(odmas)
            d = pltpu.make_async_copy(
                srcbuf.at[pl.ds(c * CH, CH), :],
                out_ref.at[pl.ds(qidx * Q + c * CH, CH), :],
                osem.at[slot],
            )
            d.start()
            odmas.append(d)

        def gather_chunk(c):
            def one(i, cnt):
                r = r_smem[q0 + c * CH + i]
                in_half = (r >= 0) & (r < VSH)
                rc = jnp.clip(r, 0, VSH - 1)

                @pl.when(in_half)
                def _():
                    pltpu.make_async_copy(
                        e_ref.at[pl.ds(rc, 1), :],
                        gbuf.at[pl.ds(c * CH + i, 1), :],
                        gsem.at[c],
                    ).start()

                return cnt + in_half.astype(jnp.int32)

            def two(k, cnt):
                cnt = one(2 * k, cnt)
                return one(2 * k + 1, cnt)

            return lax.fori_loop(0, CH // 2, two, 0)

        def send_x(c, nc):
            def drain(i, _):
                pltpu.make_async_copy(
                    e_ref.at[pl.ds(0, 1), :], gbuf.at[pl.ds(0, 1), :],
                    gsem.at[c],
                ).wait()
                return 0

            lax.fori_loop(0, nc, drain, 0)
            sl = pl.ds(c * CH, CH)
            pbuf[sl, :] = gbuf[sl, :].astype(jnp.bfloat16)
            r = pltpu.make_async_remote_copy(
                src_ref=pbuf.at[sl, :], dst_ref=xbuf.at[sl, :],
                send_sem=sx.at[c], recv_sem=rx.at[c],
                device_id=px, device_id_type=pl.DeviceIdType.MESH,
            )
            r.start()
            return r

        def complete(c):
            sl = pl.ds(c * CH, CH)
            rdx[c].wait_recv()
            cbuf[sl, :] = pbuf[sl, :] + xbuf[sl, :]
            ry_ = pltpu.make_async_remote_copy(
                src_ref=cbuf.at[sl, :], dst_ref=ybuf.at[sl, :],
                send_sem=sy.at[c], recv_sem=ry.at[c],
                device_id=py, device_id_type=pl.DeviceIdType.MESH,
            )
            ry_.start()
            rz_ = pltpu.make_async_remote_copy(
                src_ref=cbuf.at[sl, :], dst_ref=zbuf.at[sl, :],
                send_sem=sz.at[c], recv_sem=rz.at[c],
                device_id=pz, device_id_type=pl.DeviceIdType.MESH,
            )
            rz_.start()
            rdy.append(ry_)
            rdz.append(rz_)
            store_out(myq, c, cbuf)

        def recv_z(c):
            sl = pl.ds(c * CH, CH)
            rdz[c].wait_recv()
            if c < 2:
                r = pltpu.make_async_remote_copy(
                    src_ref=zbuf.at[sl, :], dst_ref=dbuf.at[sl, :],
                    send_sem=sfy.at[c], recv_sem=rfy.at[c],
                    device_id=py, device_id_type=pl.DeviceIdType.MESH,
                )
                r.start()
                fw.append(r)
            store_out(zq, c, zbuf)

        def recv_y(c):
            sl = pl.ds(c * CH, CH)
            rdy[c].wait_recv()
            if c >= 2:
                r = pltpu.make_async_remote_copy(
                    src_ref=ybuf.at[sl, :], dst_ref=dbuf.at[sl, :],
                    send_sem=sfz.at[c - 2], recv_sem=rfz.at[c - 2],
                    device_id=pz, device_id_type=pl.DeviceIdType.MESH,
                )
                r.start()
                fw.append(r)
            store_out(yq, c, ybuf)

        def recv_d(k):
            sl = pl.ds(k * CH, CH)
            fw[k].wait_recv()
            store_out(dq, k, dbuf)

        rdx, rdy, rdz, fw = [], [], [], []

        n0 = gather_chunk(0)
        pl.semaphore_wait(barrier, 3)
        rdx.append(send_x(0, n0))
        n1 = gather_chunk(1)
        rdx.append(send_x(1, n1))
        n2 = gather_chunk(2)
        complete(0)
        rdx.append(send_x(2, n2))
        n3 = gather_chunk(3)
        complete(1)
        rdx.append(send_x(3, n3))
        recv_z(0)
        complete(2)
        recv_y(0)
        recv_z(1)
        complete(3)
        recv_y(1)
        recv_z(2)
        recv_y(2)
        recv_z(3)
        recv_y(3)
        recv_d(0)
        recv_d(1)
        recv_d(2)
        recv_d(3)

        for r in rdx + rdy + rdz + fw:
            r.wait_send()
        for d in odmas:
            d.wait()

    return pl.pallas_call(
        body,
        out_shape=jax.ShapeDtypeStruct((T, D), jnp.bfloat16),
        in_specs=[
            pl.BlockSpec(memory_space=pltpu.SMEM),
            pl.BlockSpec(memory_space=pl.ANY),
        ],
        out_specs=pl.BlockSpec(memory_space=pl.ANY),
        scratch_shapes=[
            pltpu.VMEM((Q, D), jnp.float32),
            pltpu.VMEM((Q, D), jnp.bfloat16),
            pltpu.VMEM((Q, D), jnp.bfloat16),
            pltpu.VMEM((Q, D), jnp.bfloat16),
            pltpu.VMEM((Q, D), jnp.bfloat16),
            pltpu.VMEM((Q, D), jnp.bfloat16),
            pltpu.VMEM((Q, D), jnp.bfloat16),
            pltpu.SemaphoreType.DMA((C,)),
            pltpu.SemaphoreType.DMA((C,)),
            pltpu.SemaphoreType.DMA((C,)),
            pltpu.SemaphoreType.DMA((C,)),
            pltpu.SemaphoreType.DMA((C,)),
            pltpu.SemaphoreType.DMA((C,)),
            pltpu.SemaphoreType.DMA((C,)),
            pltpu.SemaphoreType.DMA((2,)),
            pltpu.SemaphoreType.DMA((2,)),
            pltpu.SemaphoreType.DMA((2,)),
            pltpu.SemaphoreType.DMA((2,)),
            pltpu.SemaphoreType.DMA((4 * C,)),
        ],
        compiler_params=pltpu.CompilerParams(collective_id=0),
    )(rpre, E)


# device time: 35484 ns/iter; 1.1589x vs baseline; 1.1210x over previous
import jax
import jax.numpy as jnp
from jax import lax
from jax.experimental import pallas as pl
from jax.experimental.pallas import tpu as pltpu

T = 2048
D = 1024
VSH = 16384
Q = 512
C = 8
CH = Q // C


def kernel(ids, E):
    my_x = lax.axis_index("x")
    rpre = ids - my_x * VSH

    def body(r_smem, e_ref, out_ref,
             gbuf, pbuf, xbuf, cbuf, ybuf, zbuf, dbuf,
             gsem, sx, rx, sy, ry, sz, rz, sfy, rfy, sfz, rfz, osem):
        x = lax.axis_index("x")
        y = lax.axis_index("y")
        z = lax.axis_index("z")
        px = (1 - x, y, z)
        py = (x, 1 - y, z)
        pz = (x, y, 1 - z)
        myq = 2 * y + z
        yq = 2 * (1 - y) + z
        zq = 2 * y + (1 - z)
        dq = 2 * (1 - y) + (1 - z)
        q0 = myq * Q

        barrier = pltpu.get_barrier_semaphore()
        for nbr in (px, py, pz):
            pl.semaphore_signal(
                barrier, inc=1, device_id=nbr,
                device_id_type=pl.DeviceIdType.MESH,
            )

        gbuf[...] = jnp.zeros((Q, D), jnp.float32)

        odmas = []

        def store_out(qidx, c, srcbuf):
            slot = len(odmas)
            d = pltpu.make_async_copy(
                srcbuf.at[pl.ds(c * CH, CH), :],
                out_ref.at[pl.ds(qidx * Q + c * CH, CH), :],
                osem.at[slot],
            )
            d.start()
            odmas.append(d)

        def gather_chunk(c):
            def one(i, cnt):
                r = r_smem[q0 + c * CH + i]
                in_half = (r >= 0) & (r < VSH)
                rc = jnp.clip(r, 0, VSH - 1)

                @pl.when(in_half)
                def _():
                    pltpu.make_async_copy(
                        e_ref.at[pl.ds(rc, 1), :],
                        gbuf.at[pl.ds(c * CH + i, 1), :],
                        gsem.at[c],
                    ).start()

                return cnt + in_half.astype(jnp.int32)

            def two(k, cnt):
                cnt = one(2 * k, cnt)
                return one(2 * k + 1, cnt)

            return lax.fori_loop(0, CH // 2, two, 0)

        def send_x(c, nc):
            def drain(i, _):
                pltpu.make_async_copy(
                    e_ref.at[pl.ds(0, 1), :], gbuf.at[pl.ds(0, 1), :],
                    gsem.at[c],
                ).wait()
                return 0

            lax.fori_loop(0, nc, drain, 0)
            sl = pl.ds(c * CH, CH)
            pbuf[sl, :] = gbuf[sl, :].astype(jnp.bfloat16)
            r = pltpu.make_async_remote_copy(
                src_ref=pbuf.at[sl, :], dst_ref=xbuf.at[sl, :],
                send_sem=sx.at[c], recv_sem=rx.at[c],
                device_id=px, device_id_type=pl.DeviceIdType.MESH,
            )
            r.start()
            return r

        def complete(c):
            sl = pl.ds(c * CH, CH)
            rdx[c].wait_recv()
            cbuf[sl, :] = pbuf[sl, :] + xbuf[sl, :]
            ry_ = pltpu.make_async_remote_copy(
                src_ref=cbuf.at[sl, :], dst_ref=ybuf.at[sl, :],
                send_sem=sy.at[c], recv_sem=ry.at[c],
                device_id=py, device_id_type=pl.DeviceIdType.MESH,
            )
            ry_.start()
            rz_ = pltpu.make_async_remote_copy(
                src_ref=cbuf.at[sl, :], dst_ref=zbuf.at[sl, :],
                send_sem=sz.at[c], recv_sem=rz.at[c],
                device_id=pz, device_id_type=pl.DeviceIdType.MESH,
            )
            rz_.start()
            rdy.append(ry_)
            rdz.append(rz_)
            store_out(myq, c, cbuf)

        def recv_z(c):
            sl = pl.ds(c * CH, CH)
            rdz[c].wait_recv()
            if c < C // 2:
                r = pltpu.make_async_remote_copy(
                    src_ref=zbuf.at[sl, :], dst_ref=dbuf.at[sl, :],
                    send_sem=sfy.at[c], recv_sem=rfy.at[c],
                    device_id=py, device_id_type=pl.DeviceIdType.MESH,
                )
                r.start()
                fw.append(r)
            store_out(zq, c, zbuf)

        def recv_y(c):
            sl = pl.ds(c * CH, CH)
            rdy[c].wait_recv()
            if c >= C // 2:
                r = pltpu.make_async_remote_copy(
                    src_ref=ybuf.at[sl, :], dst_ref=dbuf.at[sl, :],
                    send_sem=sfz.at[c - C // 2], recv_sem=rfz.at[c - C // 2],
                    device_id=pz, device_id_type=pl.DeviceIdType.MESH,
                )
                r.start()
                fw.append(r)
            store_out(yq, c, ybuf)

        def recv_d(k):
            sl = pl.ds(k * CH, CH)
            fw[k].wait_recv()
            store_out(dq, k, dbuf)

        rdx, rdy, rdz, fw = [], [], [], []

        for c in range(C):
            nc = gather_chunk(c)
            if c == 0:
                pl.semaphore_wait(barrier, 3)
            if c >= 2:
                complete(c - 2)
            rdx.append(send_x(c, nc))
        pending = list(range(C - 2, C))
        for c in range(C):
            recv_z(c)
            if pending:
                complete(pending.pop(0))
            recv_y(c)
        for k in range(C):
            recv_d(k)

        for r in rdx + rdy + rdz + fw:
            r.wait_send()
        for d in odmas:
            d.wait()

    return pl.pallas_call(
        body,
        out_shape=jax.ShapeDtypeStruct((T, D), jnp.bfloat16),
        in_specs=[
            pl.BlockSpec(memory_space=pltpu.SMEM),
            pl.BlockSpec(memory_space=pl.ANY),
        ],
        out_specs=pl.BlockSpec(memory_space=pl.ANY),
        scratch_shapes=[
            pltpu.VMEM((Q, D), jnp.float32),
            pltpu.VMEM((Q, D), jnp.bfloat16),
            pltpu.VMEM((Q, D), jnp.bfloat16),
            pltpu.VMEM((Q, D), jnp.bfloat16),
            pltpu.VMEM((Q, D), jnp.bfloat16),
            pltpu.VMEM((Q, D), jnp.bfloat16),
            pltpu.VMEM((Q, D), jnp.bfloat16),
            pltpu.SemaphoreType.DMA((C,)),
            pltpu.SemaphoreType.DMA((C,)),
            pltpu.SemaphoreType.DMA((C,)),
            pltpu.SemaphoreType.DMA((C,)),
            pltpu.SemaphoreType.DMA((C,)),
            pltpu.SemaphoreType.DMA((C,)),
            pltpu.SemaphoreType.DMA((C,)),
            pltpu.SemaphoreType.DMA((C // 2,)),
            pltpu.SemaphoreType.DMA((C // 2,)),
            pltpu.SemaphoreType.DMA((C // 2,)),
            pltpu.SemaphoreType.DMA((C // 2,)),
            pltpu.SemaphoreType.DMA((4 * C,)),
        ],
        compiler_params=pltpu.CompilerParams(collective_id=0),
    )(rpre, E)
